# baseline (speedup 1.0000x reference)
import functools

import jax
import jax.numpy as jnp
from jax import lax
from jax.experimental import pallas as pl
from jax.experimental.pallas import tpu as pltpu
from jax.experimental.pallas import tpu_sc as plsc

E = 160000
D = 256
L = 16
NC = 2
NS = 16
NW = NC * NS
NCOMBO = 60
ATT_ROW = 60
TROWS = 61

MAIN_PER_W = 4992
NG = MAIN_PER_W // L
SB_G = 4
SB_E = SB_G * L
NSB = NG // SB_G
TAIL_BASE = NW * MAIN_PER_W

_mesh = plsc.VectorSubcoreMesh(core_axis_name="c", subcore_axis_name="s")


@functools.partial(
    pl.kernel,
    mesh=_mesh,
    compiler_params=pltpu.CompilerParams(needs_layout_passes=False),
    out_type=jax.ShapeDtypeStruct((E, D), jnp.float32),
    scratch_types=[
        pltpu.VMEM((MAIN_PER_W * 3,), jnp.int32),
        pltpu.VMEM((5 * D,), jnp.float32),
        pltpu.VMEM((6 * D,), jnp.float32),
        pltpu.VMEM((2 * D,), jnp.float32),
        pltpu.VMEM((D,), jnp.float32),
        pltpu.VMEM((TROWS * D,), jnp.float32),
        pltpu.VMEM((SB_E, D), jnp.float32),
        pltpu.VMEM((SB_E, D), jnp.float32),
        pltpu.VMEM((SB_E, D), jnp.float32),
        pltpu.VMEM((SB_E, D), jnp.float32),
        pltpu.SemaphoreType.DMA,
        pltpu.SemaphoreType.DMA,
        pltpu.SemaphoreType.DMA,
        pltpu.SemaphoreType.DMA,
    ],
)
def _be_kernel(xf, w0f, w1f, w2f, attf, out,
               xv, w0v, w1v, w2v, attv, tvv, stg0, stg1, stg2, stg3,
               osem0, osem1, osem2, osem3):
    cid = lax.axis_index("c")
    sid = lax.axis_index("s")
    wid = sid * NC + cid
    base_e = wid * MAIN_PER_W

    pltpu.sync_copy(xf.at[pl.ds(base_e * 3, MAIN_PER_W * 3)], xv)
    pltpu.sync_copy(w0f, w0v)
    pltpu.sync_copy(w1f, w1v)
    pltpu.sync_copy(w2f, w2v)
    pltpu.sync_copy(attf, attv)

    iota = lax.iota(jnp.int32, L)

    def vfull(val):
        return jnp.full((L,), val, jnp.int32)

    for r in range(NCOMBO):
        i0 = r // 12
        i1 = (r // 2) % 6
        i2 = r % 2

        def rb_body(j, _, i0=i0, i1=i1, i2=i2, r=r):
            off = jnp.full((L,), j * L, jnp.int32) + iota
            v = (plsc.load_gather(w0v, [off + vfull(i0 * D)])
                 + plsc.load_gather(w1v, [off + vfull(i1 * D)])
                 + plsc.load_gather(w2v, [off + vfull(i2 * D)]))
            plsc.store_scatter(tvv, [off + vfull(r * D)], v)
            return 0

        lax.fori_loop(0, D // L, rb_body, 0)
    for j in range(D // L):
        tvv[pl.ds(ATT_ROW * D + j * L, L)] = attv[pl.ds(j * L, L)]

    i3 = iota * vfull(3)
    cmask = vfull(D - 1)

    def _rowbase(off_vec):
        x0 = plsc.load_gather(xv, [i3 + off_vec])
        x1 = plsc.load_gather(xv, [i3 + off_vec + vfull(1)])
        x2 = plsc.load_gather(xv, [i3 + off_vec + vfull(2)])
        cidx = x0 * vfull(12) + x1 * vfull(2) + x2
        cidx = jnp.where(x0 < vfull(0), vfull(ATT_ROW), cidx)
        return cidx * vfull(D)

    rowv = [vfull(g * L) + iota for g in range(SB_G)]

    def _fill_stage(sb, stg):
        base3 = sb * (SB_E * 3)
        rowbases = [_rowbase(jnp.full((L,), base3 + g * (3 * L), jnp.int32))
                    for g in range(SB_G)]

        @plsc.parallel_loop(0, D, unroll=8, carry=iota)
        def col_body(c, colv):
            vals = [plsc.load_gather(tvv, [rowbases[g] + colv])
                    for g in range(SB_G)]
            for g in range(SB_G):
                plsc.store_scatter(stg, [rowv[g], colv], vals[g])
            return (colv + vfull(1)) & cmask

    bufs = ((stg0, osem0), (stg1, osem1), (stg2, osem2), (stg3, osem3))
    NBUF = 4

    def _run_block(b, sb):
        stg, osem = bufs[b]
        _fill_stage(sb, stg)
        pltpu.make_async_copy(
            stg, out.at[pl.ds(base_e + sb * SB_E, SB_E)], osem).start()

    def _wait_out(b):
        stg, osem = bufs[b]
        pltpu.make_async_copy(stg, out.at[pl.ds(base_e, SB_E)], osem).wait()

    for k in range(NBUF):
        _run_block(k, k)

    NLOOP = (NSB - NBUF - 2) // NBUF

    def ring_body(p, _):
        for k in range(NBUF):
            _wait_out(k)
            _run_block(k, NBUF + p * NBUF + k)
        return 0

    lax.fori_loop(0, NLOOP, ring_body, 0)

    _wait_out(0)
    _run_block(0, NSB - 2)
    _wait_out(1)
    _run_block(1, NSB - 1)

    _wait_out(2)
    _wait_out(3)
    _wait_out(0)
    _wait_out(1)

    @pl.when(wid < 16)
    def _tail():
        tb = TAIL_BASE + wid * L
        pltpu.sync_copy(xf.at[pl.ds(tb * 3, 3 * L)], xv.at[pl.ds(0, 3 * L)])
        rowbase = _rowbase(vfull(0))

        @plsc.parallel_loop(0, D, unroll=8, carry=iota)
        def col_body(c, colv):
            v = plsc.load_gather(tvv, [rowbase + colv])
            plsc.store_scatter(stg0, [iota, colv], v)
            return (colv + vfull(1)) & cmask
        pltpu.sync_copy(stg0.at[pl.ds(0, L)], out.at[pl.ds(tb, L)])


def kernel(x, W0, W1, W2, att_emb):
    return _be_kernel(
        x.reshape(-1),
        W0.reshape(-1),
        W1.reshape(-1),
        W2.reshape(-1),
        att_emb,
    )


# speedup vs baseline: 1.6578x; 1.6578x over previous
import functools

import jax
import jax.numpy as jnp
from jax import lax
from jax.experimental import pallas as pl
from jax.experimental.pallas import tpu as pltpu
from jax.experimental.pallas import tpu_sc as plsc

E = 160000
D = 256
L = 16
NC = 2
NS = 16
NW = NC * NS
NCOMBO = 60
ATT_ROW = 60
TROWS = 61

MAIN_PER_W = 4992
NG = MAIN_PER_W // L
SB_G = 4
SB_E = SB_G * L
NSB = NG // SB_G
TAIL_BASE = NW * MAIN_PER_W

_mesh = plsc.VectorSubcoreMesh(core_axis_name="c", subcore_axis_name="s")


@functools.partial(
    pl.kernel,
    mesh=_mesh,
    compiler_params=pltpu.CompilerParams(needs_layout_passes=False),
    out_type=jax.ShapeDtypeStruct((E, D), jnp.float32),
    scratch_types=[
        pltpu.VMEM((MAIN_PER_W,), jnp.int32),
        pltpu.VMEM((5 * D,), jnp.float32),
        pltpu.VMEM((6 * D,), jnp.float32),
        pltpu.VMEM((2 * D,), jnp.float32),
        pltpu.VMEM((D,), jnp.float32),
        pltpu.VMEM((TROWS * D,), jnp.float32),
        pltpu.VMEM((SB_E, D), jnp.float32),
        pltpu.VMEM((SB_E, D), jnp.float32),
        pltpu.VMEM((SB_E, D), jnp.float32),
        pltpu.VMEM((SB_E, D), jnp.float32),
        pltpu.SemaphoreType.DMA,
        pltpu.SemaphoreType.DMA,
        pltpu.SemaphoreType.DMA,
        pltpu.SemaphoreType.DMA,
    ],
)
def _be_kernel(xf, w0f, w1f, w2f, attf, out,
               xv, w0v, w1v, w2v, attv, tvv, stg0, stg1, stg2, stg3,
               osem0, osem1, osem2, osem3):
    cid = lax.axis_index("c")
    sid = lax.axis_index("s")
    wid = sid * NC + cid
    base_e = wid * MAIN_PER_W

    pltpu.sync_copy(xf.at[pl.ds(base_e, MAIN_PER_W)], xv)
    pltpu.sync_copy(w0f, w0v)
    pltpu.sync_copy(w1f, w1v)
    pltpu.sync_copy(w2f, w2v)
    pltpu.sync_copy(attf, attv)

    iota = lax.iota(jnp.int32, L)

    def vfull(val):
        return jnp.full((L,), val, jnp.int32)

    for r in range(NCOMBO):
        i0 = r // 12
        i1 = (r // 2) % 6
        i2 = r % 2

        def rb_body(j, _, i0=i0, i1=i1, i2=i2, r=r):
            off = jnp.full((L,), j * L, jnp.int32) + iota
            v = (plsc.load_gather(w0v, [off + vfull(i0 * D)])
                 + plsc.load_gather(w1v, [off + vfull(i1 * D)])
                 + plsc.load_gather(w2v, [off + vfull(i2 * D)]))
            plsc.store_scatter(tvv, [off + vfull(r * D)], v)
            return 0

        lax.fori_loop(0, D // L, rb_body, 0)
    for j in range(D // L):
        tvv[pl.ds(ATT_ROW * D + j * L, L)] = attv[pl.ds(j * L, L)]

    cmask = vfull(D - 1)

    rowv = [vfull(g * L) + iota for g in range(SB_G)]

    def _fill_stage(sb, stg):
        rowbases = [xv[pl.ds(sb * SB_E + g * L, L)] * vfull(D)
                    for g in range(SB_G)]

        @plsc.parallel_loop(0, D, unroll=8, carry=iota)
        def col_body(c, colv):
            vals = [plsc.load_gather(tvv, [rowbases[g] + colv])
                    for g in range(SB_G)]
            for g in range(SB_G):
                plsc.store_scatter(stg, [rowv[g], colv], vals[g])
            return (colv + vfull(1)) & cmask

    bufs = ((stg0, osem0), (stg1, osem1), (stg2, osem2), (stg3, osem3))
    NBUF = 4

    def _run_block(b, sb):
        stg, osem = bufs[b]
        _fill_stage(sb, stg)
        pltpu.make_async_copy(
            stg, out.at[pl.ds(base_e + sb * SB_E, SB_E)], osem).start()

    def _wait_out(b):
        stg, osem = bufs[b]
        pltpu.make_async_copy(stg, out.at[pl.ds(base_e, SB_E)], osem).wait()

    for k in range(NBUF):
        _run_block(k, k)

    NLOOP = (NSB - NBUF - 2) // NBUF

    def ring_body(p, _):
        for k in range(NBUF):
            _wait_out(k)
            _run_block(k, NBUF + p * NBUF + k)
        return 0

    lax.fori_loop(0, NLOOP, ring_body, 0)

    _wait_out(0)
    _run_block(0, NSB - 2)
    _wait_out(1)
    _run_block(1, NSB - 1)

    _wait_out(2)
    _wait_out(3)
    _wait_out(0)
    _wait_out(1)

    @pl.when(wid < 16)
    def _tail():
        tb = TAIL_BASE + wid * L
        pltpu.sync_copy(xf.at[pl.ds(tb, L)], xv.at[pl.ds(0, L)])
        rowbase = xv[pl.ds(0, L)] * vfull(D)

        @plsc.parallel_loop(0, D, unroll=8, carry=iota)
        def col_body(c, colv):
            v = plsc.load_gather(tvv, [rowbase + colv])
            plsc.store_scatter(stg0, [iota, colv], v)
            return (colv + vfull(1)) & cmask
        pltpu.sync_copy(stg0.at[pl.ds(0, L)], out.at[pl.ds(tb, L)])


def kernel(x, W0, W1, W2, att_emb):
    x0 = x[:, 0]
    cidx = jnp.where(x0 < 0, ATT_ROW, x0 * 12 + x[:, 1] * 2 + x[:, 2])
    return _be_kernel(
        cidx.astype(jnp.int32),
        W0.reshape(-1),
        W1.reshape(-1),
        W2.reshape(-1),
        att_emb,
    )
